# baseline (speedup 1.0000x reference)
import functools

import jax
import jax.numpy as jnp
from jax import lax
from jax.experimental import pallas as pl
from jax.experimental.pallas import tpu as pltpu
from jax.experimental.pallas import tpu_sc as plsc

BINS_ = 10
ROWS = 4096
COLS = 4096
SC_ROWS = 1024
TC_ROWS = ROWS - SC_ROWS
BLK = 128
GRID = TC_ROWS // BLK
TC_TOTAL = float(TC_ROWS * COLS)
NACC = 2 * BINS_ - 1

NW = 32
W_ROWS = SC_ROWS // NW
VPR = COLS // 16
UNROLL = 8
BANK = 2 * 16 * BINS_
TBL = UNROLL * BANK

_LOG1P_COEF = (
    9.083786844943376e-08, 0.9999914545717464, -0.49980116320372914,
    0.3313340057250358, -0.23919071732133323, 0.16478349729867933,
    -0.09231376866991943, 0.03441859352056854, -0.006074877643740236,
)


def _tc_body(x_ref, y_ref, out_ref, acc_ref):
    pid = pl.program_id(0)

    @pl.when(pid == 0)
    def _init():
        acc_ref[...] = jnp.zeros_like(acc_ref)

    zero = jnp.zeros((8, 128), jnp.float32)
    for r in range(BLK // 8):
        sge = [zero] * BINS_
        cge = [zero] * (BINS_ - 1)
        for c in range(COLS // 128):
            x = x_ref[r * 8:(r + 1) * 8, c * 128:(c + 1) * 128]
            y = y_ref[r * 8:(r + 1) * 8, c * 128:(c + 1) * 128]
            e = jnp.exp(-jnp.abs(x))
            u = 1.0 + e
            s0 = 1.0 / u
            sig = jnp.where(x >= 0.0, s0, 1.0 - s0)
            g = jnp.abs(sig - y)
            pe = jnp.maximum(x, 0.0) - x * y + jnp.log(u)
            sge[0] = sge[0] + pe
            for b in range(1, BINS_):
                mf = jnp.where(g >= float(b) / float(BINS_), 1.0, 0.0)
                sge[b] = sge[b] + pe * mf
                cge[b - 1] = cge[b - 1] + mf
        for b in range(BINS_):
            acc_ref[b] += sge[b]
        for b in range(BINS_ - 1):
            acc_ref[BINS_ + b] += cge[b]

    @pl.when(pid == GRID - 1)
    def _finish():
        for i in range(NACC):
            out_ref[i] = jnp.sum(acc_ref[i])


CS = 16384
W_CHUNKS = W_ROWS * COLS // CS


def _sc_body(x_hbm, y_hbm, out_hbm, xbuf, ybuf, tbl, sx0, sx1, sy0, sy1):
    wid = lax.axis_index("s") * 2 + lax.axis_index("c")
    base0 = (TC_ROWS + wid * W_ROWS) * COLS

    zero16 = jnp.zeros((16,), jnp.float32)
    for k in range(TBL // 16):
        tbl[k * 16:(k + 1) * 16] = zero16

    lane = lax.iota(jnp.int32, 16)
    sems_x = (sx0, sx1)
    sems_y = (sy0, sy1)

    def issue(ci, p):
        pltpu.async_copy(
            x_hbm.at[pl.ds(base0 + ci * CS, CS)], xbuf.at[p], sems_x[p])
        pltpu.async_copy(
            y_hbm.at[pl.ds(base0 + ci * CS, CS)], ybuf.at[p], sems_y[p])

    def wait(p):
        pltpu.make_async_copy(
            x_hbm.at[pl.ds(base0, CS)], xbuf.at[p], sems_x[p]).wait()
        pltpu.make_async_copy(
            y_hbm.at[pl.ds(base0, CS)], ybuf.at[p], sems_y[p]).wait()

    def compute_row(p):
        def body(j, carry):
            for k in range(UNROLL):
                off = j * (16 * UNROLL) + k * 16
                x = xbuf[p, pl.ds(off, 16)]
                y = ybuf[p, pl.ds(off, 16)]
                e = jnp.exp(-jnp.abs(x))
                u = 1.0 + e
                s0 = 1.0 / u
                sig = jnp.where(x >= 0.0, s0, 1.0 - s0)
                z = jnp.abs(sig - y) * float(BINS_)
                gid = jnp.minimum(z.astype(jnp.int32), BINS_ - 1)
                idx = gid * 16 + lane
                lp = jnp.full((16,), _LOG1P_COEF[-1], jnp.float32)
                for cf in _LOG1P_COEF[-2::-1]:
                    lp = lp * e + jnp.float32(cf)
                pe = jnp.maximum(x, 0.0) - x * y + lp
                bank = k * BANK
                plsc.addupdate_scatter(
                    tbl, [idx + bank], jnp.ones((16,), jnp.float32))
                plsc.addupdate_scatter(tbl, [idx + (bank + 16 * BINS_)], pe)
            return carry

        lax.fori_loop(0, CS // 16 // UNROLL, body, 0)

    issue(0, 0)
    issue(1, 1)

    def outer_body(i, carry):
        ci = 2 * i
        wait(0)
        compute_row(0)
        issue(ci + 2, 0)
        wait(1)
        compute_row(1)
        issue(ci + 3, 1)
        return carry

    lax.fori_loop(0, W_CHUNKS // 2 - 1, outer_body, 0)
    wait(0)
    compute_row(0)
    wait(1)
    compute_row(1)

    pltpu.sync_copy(tbl, out_hbm.at[wid])


_sc_mesh = plsc.VectorSubcoreMesh(core_axis_name="c", subcore_axis_name="s")
_sc_kernel = functools.partial(
    pl.kernel,
    mesh=_sc_mesh,
    compiler_params=pltpu.CompilerParams(needs_layout_passes=False),
    out_type=jax.ShapeDtypeStruct((NW, TBL), jnp.float32),
    scratch_types=[
        pltpu.VMEM((2, CS), jnp.float32),
        pltpu.VMEM((2, CS), jnp.float32),
        pltpu.VMEM((TBL,), jnp.float32),
        pltpu.SemaphoreType.DMA,
        pltpu.SemaphoreType.DMA,
        pltpu.SemaphoreType.DMA,
        pltpu.SemaphoreType.DMA,
    ],
)(_sc_body)


def _combine_body(tc_ref, sc_ref, out_ref):
    sc = sc_ref[...]
    pos = lax.broadcasted_iota(jnp.int32, (NW, TBL), 1) % BANK
    c_sc = []
    s_sc = []
    for b in range(BINS_):
        mc = (pos >= b * 16) & (pos < b * 16 + 16)
        ms = (pos >= 160 + b * 16) & (pos < 160 + b * 16 + 16)
        c_sc.append(jnp.sum(jnp.where(mc, sc, 0.0)))
        s_sc.append(jnp.sum(jnp.where(ms, sc, 0.0)))

    sge = [tc_ref[i] for i in range(BINS_)]
    cge = [tc_ref[BINS_ + i] for i in range(BINS_ - 1)]
    total = 0.0
    n = 0.0
    for b in range(BINS_):
        s_hi = sge[b + 1] if b + 1 < BINS_ else 0.0
        c_lo = cge[b - 1] if b >= 1 else TC_TOTAL
        c_hi = cge[b] if b < BINS_ - 1 else 0.0
        s_b = sge[b] - s_hi + s_sc[b]
        c_b = c_lo - c_hi + c_sc[b]
        nonempty = c_b > 0.0
        total += jnp.where(nonempty, s_b / jnp.where(nonempty, c_b, 1.0), 0.0)
        n += jnp.where(nonempty, 1.0, 0.0)
    out_ref[0] = total / n


def kernel(x, y):
    sc_part = _sc_kernel(x.reshape(-1), y.reshape(-1))
    tc_part = pl.pallas_call(
        _tc_body,
        grid=(GRID,),
        in_specs=[
            pl.BlockSpec((BLK, COLS), lambda i: (i, 0)),
            pl.BlockSpec((BLK, COLS), lambda i: (i, 0)),
        ],
        out_specs=pl.BlockSpec(memory_space=pltpu.SMEM),
        out_shape=jax.ShapeDtypeStruct((NACC,), jnp.float32),
        scratch_shapes=[pltpu.VMEM((NACC, 8, 128), jnp.float32)],
    )(x, y)
    out = pl.pallas_call(
        _combine_body,
        in_specs=[
            pl.BlockSpec(memory_space=pltpu.SMEM),
            pl.BlockSpec((NW, TBL), lambda: (0, 0)),
        ],
        out_specs=pl.BlockSpec(memory_space=pltpu.SMEM),
        out_shape=jax.ShapeDtypeStruct((1,), jnp.float32),
    )(tc_part, sc_part)
    return out[0]


# speedup vs baseline: 1.5143x; 1.5143x over previous
import functools

import jax
import jax.numpy as jnp
from jax import lax
from jax.experimental import pallas as pl
from jax.experimental.pallas import tpu as pltpu
from jax.experimental.pallas import tpu_sc as plsc

BINS_ = 10
ROWS = 4096
COLS = 4096
SC_ROWS = 512
TC_ROWS = ROWS - SC_ROWS
BLK = 128
GRID = TC_ROWS // BLK
TC_TOTAL = float(TC_ROWS * COLS)
NACC = 2 * BINS_ - 1

NW = 32
W_ROWS = SC_ROWS // NW
VPR = COLS // 16
UNROLL = 8
BANK = 2 * 16 * BINS_
TBL = UNROLL * BANK

_LOG1P_COEF = (
    9.083786844943376e-08, 0.9999914545717464, -0.49980116320372914,
    0.3313340057250358, -0.23919071732133323, 0.16478349729867933,
    -0.09231376866991943, 0.03441859352056854, -0.006074877643740236,
)


def _tc_body(x_ref, y_ref, out_ref, acc_ref):
    pid = pl.program_id(0)

    @pl.when(pid == 0)
    def _init():
        acc_ref[...] = jnp.zeros_like(acc_ref)

    zero = jnp.zeros((8, 128), jnp.float32)
    for r in range(BLK // 8):
        sge = [zero] * BINS_
        cge = [zero] * (BINS_ - 1)
        for c in range(COLS // 128):
            x = x_ref[r * 8:(r + 1) * 8, c * 128:(c + 1) * 128]
            y = y_ref[r * 8:(r + 1) * 8, c * 128:(c + 1) * 128]
            e = jnp.exp(-jnp.abs(x))
            u = 1.0 + e
            s0 = 1.0 / u
            sig = jnp.where(x >= 0.0, s0, 1.0 - s0)
            g = jnp.abs(sig - y)
            pe = jnp.maximum(x, 0.0) - x * y + jnp.log(u)
            sge[0] = sge[0] + pe
            for b in range(1, BINS_):
                mf = jnp.where(g >= float(b) / float(BINS_), 1.0, 0.0)
                sge[b] = sge[b] + pe * mf
                cge[b - 1] = cge[b - 1] + mf
        for b in range(BINS_):
            acc_ref[b] += sge[b]
        for b in range(BINS_ - 1):
            acc_ref[BINS_ + b] += cge[b]

    @pl.when(pid == GRID - 1)
    def _finish():
        for i in range(NACC):
            out_ref[i] = jnp.sum(acc_ref[i])


CS = 4096
W_CHUNKS = W_ROWS * COLS // CS


def _sc_body(x_hbm, y_hbm, out_hbm, xbuf, ybuf, tbl, sx0, sx1, sy0, sy1):
    wid = lax.axis_index("s") * 2 + lax.axis_index("c")
    base0 = (TC_ROWS + wid * W_ROWS) * COLS

    zero16 = jnp.zeros((16,), jnp.float32)
    for k in range(TBL // 16):
        tbl[k * 16:(k + 1) * 16] = zero16

    lane = lax.iota(jnp.int32, 16)
    sems_x = (sx0, sx1)
    sems_y = (sy0, sy1)

    def issue(ci, p):
        pltpu.async_copy(
            x_hbm.at[pl.ds(base0 + ci * CS, CS)], xbuf.at[p], sems_x[p])
        pltpu.async_copy(
            y_hbm.at[pl.ds(base0 + ci * CS, CS)], ybuf.at[p], sems_y[p])

    def wait(p):
        pltpu.make_async_copy(
            x_hbm.at[pl.ds(base0, CS)], xbuf.at[p], sems_x[p]).wait()
        pltpu.make_async_copy(
            y_hbm.at[pl.ds(base0, CS)], ybuf.at[p], sems_y[p]).wait()

    def compute_row(p):
        def body(j, carry):
            for k in range(UNROLL):
                off = j * (16 * UNROLL) + k * 16
                x = xbuf[p, pl.ds(off, 16)]
                y = ybuf[p, pl.ds(off, 16)]
                e = jnp.exp(-jnp.abs(x))
                u = 1.0 + e
                s0 = 1.0 / u
                sig = jnp.where(x >= 0.0, s0, 1.0 - s0)
                z = jnp.abs(sig - y) * float(BINS_)
                gid = jnp.minimum(z.astype(jnp.int32), BINS_ - 1)
                idx = gid * 16 + lane
                lp = jnp.full((16,), _LOG1P_COEF[-1], jnp.float32)
                for cf in _LOG1P_COEF[-2::-1]:
                    lp = lp * e + jnp.float32(cf)
                pe = jnp.maximum(x, 0.0) - x * y + lp
                bank = k * BANK
                plsc.addupdate_scatter(
                    tbl, [idx + bank], jnp.ones((16,), jnp.float32))
                plsc.addupdate_scatter(tbl, [idx + (bank + 16 * BINS_)], pe)
            return carry

        lax.fori_loop(0, CS // 16 // UNROLL, body, 0)

    issue(0, 0)
    issue(1, 1)

    def outer_body(i, carry):
        ci = 2 * i
        wait(0)
        compute_row(0)
        issue(ci + 2, 0)
        wait(1)
        compute_row(1)
        issue(ci + 3, 1)
        return carry

    lax.fori_loop(0, W_CHUNKS // 2 - 1, outer_body, 0)
    wait(0)
    compute_row(0)
    wait(1)
    compute_row(1)

    pltpu.sync_copy(tbl, out_hbm.at[wid])


_sc_mesh = plsc.VectorSubcoreMesh(core_axis_name="c", subcore_axis_name="s")
_sc_kernel = functools.partial(
    pl.kernel,
    mesh=_sc_mesh,
    compiler_params=pltpu.CompilerParams(needs_layout_passes=False),
    out_type=jax.ShapeDtypeStruct((NW, TBL), jnp.float32),
    scratch_types=[
        pltpu.VMEM((2, CS), jnp.float32),
        pltpu.VMEM((2, CS), jnp.float32),
        pltpu.VMEM((TBL,), jnp.float32),
        pltpu.SemaphoreType.DMA,
        pltpu.SemaphoreType.DMA,
        pltpu.SemaphoreType.DMA,
        pltpu.SemaphoreType.DMA,
    ],
)(_sc_body)


def _combine_body(tc_ref, sc_ref, out_ref):
    sc = sc_ref[...]
    pos = lax.broadcasted_iota(jnp.int32, (NW, TBL), 1) % BANK
    c_sc = []
    s_sc = []
    for b in range(BINS_):
        mc = (pos >= b * 16) & (pos < b * 16 + 16)
        ms = (pos >= 160 + b * 16) & (pos < 160 + b * 16 + 16)
        c_sc.append(jnp.sum(jnp.where(mc, sc, 0.0)))
        s_sc.append(jnp.sum(jnp.where(ms, sc, 0.0)))

    sge = [tc_ref[i] for i in range(BINS_)]
    cge = [tc_ref[BINS_ + i] for i in range(BINS_ - 1)]
    total = 0.0
    n = 0.0
    for b in range(BINS_):
        s_hi = sge[b + 1] if b + 1 < BINS_ else 0.0
        c_lo = cge[b - 1] if b >= 1 else TC_TOTAL
        c_hi = cge[b] if b < BINS_ - 1 else 0.0
        s_b = sge[b] - s_hi + s_sc[b]
        c_b = c_lo - c_hi + c_sc[b]
        nonempty = c_b > 0.0
        total += jnp.where(nonempty, s_b / jnp.where(nonempty, c_b, 1.0), 0.0)
        n += jnp.where(nonempty, 1.0, 0.0)
    out_ref[0] = total / n


def kernel(x, y):
    sc_part = _sc_kernel(x.reshape(-1), y.reshape(-1))
    tc_part = pl.pallas_call(
        _tc_body,
        grid=(GRID,),
        in_specs=[
            pl.BlockSpec((BLK, COLS), lambda i: (i, 0)),
            pl.BlockSpec((BLK, COLS), lambda i: (i, 0)),
        ],
        out_specs=pl.BlockSpec(memory_space=pltpu.SMEM),
        out_shape=jax.ShapeDtypeStruct((NACC,), jnp.float32),
        scratch_shapes=[pltpu.VMEM((NACC, 8, 128), jnp.float32)],
    )(x, y)
    out = pl.pallas_call(
        _combine_body,
        in_specs=[
            pl.BlockSpec(memory_space=pltpu.SMEM),
            pl.BlockSpec((NW, TBL), lambda: (0, 0)),
        ],
        out_specs=pl.BlockSpec(memory_space=pltpu.SMEM),
        out_shape=jax.ShapeDtypeStruct((1,), jnp.float32),
    )(tc_part, sc_part)
    return out[0]


# speedup vs baseline: 2.7258x; 1.8000x over previous
import jax
import jax.numpy as jnp
from jax.experimental import pallas as pl
from jax.experimental.pallas import tpu as pltpu

BINS_ = 10
ROWS = 4096
COLS = 4096
BLK = 128
GRID = ROWS // BLK
TOTAL = float(ROWS * COLS)
NACC = 2 * BINS_ - 1


def _body(x_ref, y_ref, out_ref, acc_ref):
    pid = pl.program_id(0)

    @pl.when(pid == 0)
    def _init():
        acc_ref[...] = jnp.zeros_like(acc_ref)

    zero = jnp.zeros((8, 128), jnp.float32)
    for r in range(BLK // 8):
        sge = [zero] * BINS_
        cge = [zero] * (BINS_ - 1)
        for c in range(COLS // 128):
            x = x_ref[r * 8:(r + 1) * 8, c * 128:(c + 1) * 128]
            y = y_ref[r * 8:(r + 1) * 8, c * 128:(c + 1) * 128]
            e = jnp.exp(-jnp.abs(x))
            u = 1.0 + e
            s0 = 1.0 / u
            sig = jnp.where(x >= 0.0, s0, 1.0 - s0)
            g = jnp.abs(sig - y)
            pe = jnp.maximum(x, 0.0) - x * y + jnp.log(u)
            sge[0] = sge[0] + pe
            for b in range(1, BINS_):
                mf = jnp.where(g >= float(b) / float(BINS_), 1.0, 0.0)
                sge[b] = sge[b] + pe * mf
                cge[b - 1] = cge[b - 1] + mf

        for b in range(BINS_):
            acc_ref[b] += sge[b]
        for b in range(BINS_ - 1):
            acc_ref[BINS_ + b] += cge[b]

    @pl.when(pid == GRID - 1)
    def _finish():
        s = [jnp.sum(acc_ref[b]) for b in range(BINS_)]
        cg = [jnp.sum(acc_ref[BINS_ + b]) for b in range(BINS_ - 1)]
        total = 0.0
        n = 0.0
        for b in range(BINS_):
            s_hi = s[b + 1] if b + 1 < BINS_ else 0.0
            c_lo = cg[b - 1] if b >= 1 else TOTAL
            c_hi = cg[b] if b < BINS_ - 1 else 0.0
            s_b = s[b] - s_hi
            c_b = c_lo - c_hi
            nonempty = c_b > 0.0
            total += jnp.where(nonempty, s_b / jnp.where(nonempty, c_b, 1.0), 0.0)
            n += jnp.where(nonempty, 1.0, 0.0)
        out_ref[0] = total / n


def kernel(x, y):
    out = pl.pallas_call(
        _body,
        grid=(GRID,),
        in_specs=[
            pl.BlockSpec((BLK, COLS), lambda i: (i, 0)),
            pl.BlockSpec((BLK, COLS), lambda i: (i, 0)),
        ],
        out_specs=pl.BlockSpec(memory_space=pltpu.SMEM),
        out_shape=jax.ShapeDtypeStruct((1,), jnp.float32),
        scratch_shapes=[pltpu.VMEM((NACC, 8, 128), jnp.float32)],
    )(x, y)
    return out[0]
